# baseline (speedup 1.0000x reference)
import functools

import jax
import jax.numpy as jnp
from jax import lax
from jax.experimental import pallas as pl
from jax.experimental.pallas import tpu as pltpu
from jax.experimental.pallas import tpu_sc as plsc

NUM_WORKERS = 32
CHUNK = 512


def _make_gather(n_idx: int, embed: int):
    assert n_idx % (NUM_WORKERS * CHUNK) == 0
    per_w = n_idx // NUM_WORKERS
    n_chunks = per_w // CHUNK

    mesh = plsc.VectorSubcoreMesh(core_axis_name="c", subcore_axis_name="s")

    @functools.partial(
        pl.kernel,
        mesh=mesh,
        out_type=jax.ShapeDtypeStruct((n_idx, embed), jnp.float32),
        scratch_types=[
            pltpu.VMEM((CHUNK,), jnp.int32),
            pltpu.VMEM((CHUNK, embed), jnp.float32),
            pltpu.SemaphoreType.DMA,
        ],
        compiler_params=pltpu.CompilerParams(use_tc_tiling_on_sc=False),
    )
    def gather_kernel(idx_hbm, table_hbm, out_hbm, idx_v, rows_v, sem):
        wid = lax.axis_index("s") * 2 + lax.axis_index("c")
        base = wid * per_w

        def body(i, carry):
            off = base + i * CHUNK
            pltpu.sync_copy(idx_hbm.at[pl.ds(off, CHUNK)], idx_v)
            pltpu.async_copy(table_hbm.at[idx_v], rows_v, sem).wait()
            pltpu.sync_copy(rows_v, out_hbm.at[pl.ds(off, CHUNK)])
            return carry

        lax.fori_loop(0, n_chunks, body, 0)

    return gather_kernel


def kernel(q, W):
    b, l = q.shape
    _, embed = W.shape
    flat_idx = q.reshape(b * l).astype(jnp.int32)
    out = _make_gather(b * l, embed)(flat_idx, W)
    return out.reshape(b, l, embed)


# speedup vs baseline: 1.0367x; 1.0367x over previous
import functools

import jax
import jax.numpy as jnp
from jax import lax
from jax.experimental import pallas as pl
from jax.experimental.pallas import tpu as pltpu
from jax.experimental.pallas import tpu_sc as plsc

NUM_WORKERS = 32
CHUNK = 256
NBUF = 4


def _make_gather(n_idx: int, embed: int):
    assert n_idx % (NUM_WORKERS * CHUNK * NBUF) == 0
    per_w = n_idx // NUM_WORKERS
    n_chunks = per_w // CHUNK
    n_groups = n_chunks // NBUF

    mesh = plsc.VectorSubcoreMesh(core_axis_name="c", subcore_axis_name="s")

    @functools.partial(
        pl.kernel,
        mesh=mesh,
        out_type=jax.ShapeDtypeStruct((n_idx, embed), jnp.float32),
        scratch_types=[
            pltpu.VMEM((per_w,), jnp.int32),
            pltpu.VMEM((NBUF, CHUNK, embed), jnp.float32),
            pltpu.SemaphoreType.DMA((NBUF,)),
            pltpu.SemaphoreType.DMA((NBUF,)),
        ],
        compiler_params=pltpu.CompilerParams(use_tc_tiling_on_sc=False),
    )
    def gather_kernel(idx_hbm, table_hbm, out_hbm, idx_v, rows_v, gsem, wsem):
        wid = lax.axis_index("s") * 2 + lax.axis_index("c")
        base = wid * per_w

        pltpu.sync_copy(idx_hbm.at[pl.ds(base, per_w)], idx_v)

        def gather_start(b, chunk):
            pltpu.async_copy(
                table_hbm.at[idx_v.at[pl.ds(chunk * CHUNK, CHUNK)]],
                rows_v.at[b],
                gsem.at[b],
            )

        def write_start(b, chunk):
            pltpu.async_copy(
                rows_v.at[b],
                out_hbm.at[pl.ds(base + chunk * CHUNK, CHUNK)],
                wsem.at[b],
            )

        def drain(sem, b, shape_ref):
            pltpu.make_async_copy(shape_ref, shape_ref, sem.at[b]).wait()

        for b in range(NBUF):
            gather_start(b, b)

        def body(j, carry):
            i0 = j * NBUF
            for b in range(NBUF):
                pltpu.make_async_copy(
                    table_hbm.at[idx_v.at[pl.ds(0, CHUNK)]],
                    rows_v.at[b],
                    gsem.at[b],
                ).wait()
                write_start(b, i0 + b)

            @pl.when(j + 1 < n_groups)
            def _():
                for b in range(NBUF):
                    pltpu.make_async_copy(
                        rows_v.at[b],
                        out_hbm.at[pl.ds(base, CHUNK)],
                        wsem.at[b],
                    ).wait()
                    gather_start(b, i0 + NBUF + b)

            return carry

        lax.fori_loop(0, n_groups, body, 0)

        for b in range(NBUF):
            pltpu.make_async_copy(
                rows_v.at[b],
                out_hbm.at[pl.ds(base, CHUNK)],
                wsem.at[b],
            ).wait()

    return gather_kernel


def kernel(q, W):
    b, l = q.shape
    _, embed = W.shape
    flat_idx = q.reshape(b * l).astype(jnp.int32)
    out = _make_gather(b * l, embed)(flat_idx, W)
    return out.reshape(b, l, embed)
